# baseline (speedup 1.0000x reference)
import functools

import jax
import jax.numpy as jnp
from jax import lax
from jax.experimental import pallas as pl
from jax.experimental.pallas import tpu as pltpu
from jax.experimental.pallas import tpu_sc as plsc

NODE_DIM = 32
NC = 2
NS = 16
NW = NC * NS
L = 16

B_TOTAL = 16384 * 50
PER_W = B_TOTAL // NW
CHUNK = 1600
N_CHUNKS = PER_W // CHUNK


def _rsqrt(s):
    i = plsc.bitcast(s, jnp.int32)
    i = jnp.int32(0x5F3759DF) - (i >> 1)
    y = plsc.bitcast(i, jnp.float32)
    h = 0.5 * s
    y = y * (1.5 - h * y * y)
    y = y * (1.5 - h * y * y)
    return y


_mesh = plsc.VectorSubcoreMesh(core_axis_name="c", subcore_axis_name="s")


@functools.partial(
    pl.kernel,
    mesh=_mesh,
    compiler_params=pltpu.CompilerParams(
        needs_layout_passes=False, use_tc_tiling_on_sc=False
    ),
    out_type=jax.ShapeDtypeStruct((B_TOTAL, NODE_DIM), jnp.float32),
    scratch_types=[
        pltpu.VMEM((CHUNK,), jnp.int32),
        pltpu.VMEM((CHUNK, NODE_DIM), jnp.float32),
        pltpu.SemaphoreType.DMA,
    ],
)
def _emb_norm(table_hbm, ix_hbm, out_hbm, idx_v, rows_v, sem):
    wid = lax.axis_index("s") * NC + lax.axis_index("c")
    base = wid * PER_W

    def chunk_body(g, carry):
        off = base + g * CHUNK
        pltpu.sync_copy(ix_hbm.at[pl.ds(off, CHUNK)], idx_v)
        pltpu.async_copy(table_hbm.at[idx_v], rows_v, sem).wait()

        def grp_body(g2, c2):
            rows16 = g2 * L + lax.iota(jnp.int32, L)
            vs = []
            s = None
            for c in range(NODE_DIM):
                colv = jnp.full((L,), c, jnp.int32)
                v = plsc.load_gather(rows_v, [rows16, colv])
                vs.append(v)
                sq = v * v
                s = sq if s is None else s + sq
            norm = s * _rsqrt(s)
            inv = 1.0 / (1e-7 + norm)
            for c in range(NODE_DIM):
                colv = jnp.full((L,), c, jnp.int32)
                plsc.store_scatter(rows_v, [rows16, colv], vs[c] * inv)
            return c2

        lax.fori_loop(0, CHUNK // L, grp_body, 0)
        pltpu.sync_copy(rows_v, out_hbm.at[pl.ds(off, CHUNK)])
        return carry

    lax.fori_loop(0, N_CHUNKS, chunk_body, 0)


def kernel(ix, table):
    ix_flat = ix.reshape(-1).astype(jnp.int32)
    out = _emb_norm(table, ix_flat)
    return out.reshape(ix.shape + (NODE_DIM,))


# speedup vs baseline: 1.2920x; 1.2920x over previous
import functools

import jax
import jax.numpy as jnp
from jax import lax
from jax.experimental import pallas as pl
from jax.experimental.pallas import tpu as pltpu
from jax.experimental.pallas import tpu_sc as plsc

NODE_DIM = 32
SEQ = 50
NC = 2
NS = 16
NW = NC * NS
L = 16

N_ROWS = 16384
ROWS_PER_W = N_ROWS // NW
R = 32
N_CHUNKS = ROWS_PER_W // R


def _rsqrt(s):
    i = plsc.bitcast(s, jnp.int32)
    i = jnp.int32(0x5F3759DF) - (i >> 1)
    y = plsc.bitcast(i, jnp.float32)
    h = 0.5 * s
    y = y * (1.5 - h * y * y)
    y = y * (1.5 - h * y * y)
    return y


_mesh = plsc.VectorSubcoreMesh(core_axis_name="c", subcore_axis_name="s")


@functools.partial(
    pl.kernel,
    mesh=_mesh,
    compiler_params=pltpu.CompilerParams(
        needs_layout_passes=False, use_tc_tiling_on_sc=False
    ),
    out_type=jax.ShapeDtypeStruct((N_ROWS, SEQ, NODE_DIM), jnp.float32),
    scratch_types=[
        pltpu.VMEM((R, SEQ), jnp.int32),
        pltpu.VMEM((R, SEQ, NODE_DIM), jnp.float32),
        pltpu.SemaphoreType.DMA,
    ],
)
def _emb_norm(table_hbm, ix_hbm, out_hbm, idx_v, rows_v, sem):
    wid = lax.axis_index("s") * NC + lax.axis_index("c")
    base_row = wid * ROWS_PER_W

    def chunk_body(g, carry):
        row0 = base_row + g * R
        pltpu.sync_copy(ix_hbm.at[pl.ds(row0, R)], idx_v)

        def fire(j, c2):
            pltpu.async_copy(table_hbm.at[idx_v.at[j]], rows_v.at[j], sem)
            return c2

        lax.fori_loop(0, R, fire, 0)

        def drain(j, c2):
            pltpu.make_async_copy(
                table_hbm.at[idx_v.at[j]], rows_v.at[j], sem
            ).wait()
            return c2

        lax.fori_loop(0, R, drain, 0)

        for i0 in range(0, R, L):
            ivec = i0 + lax.iota(jnp.int32, L)

            def grp_body(j, c2, ivec=ivec):
                jvec = jnp.full((L,), j, jnp.int32)
                vs = []
                s = None
                for c in range(NODE_DIM):
                    cvec = jnp.full((L,), c, jnp.int32)
                    v = plsc.load_gather(rows_v, [ivec, jvec, cvec])
                    vs.append(v)
                    sq = v * v
                    s = sq if s is None else s + sq
                norm = s * _rsqrt(s)
                inv = 1.0 / (1e-7 + norm)
                for c in range(NODE_DIM):
                    cvec = jnp.full((L,), c, jnp.int32)
                    plsc.store_scatter(rows_v, [ivec, jvec, cvec], vs[c] * inv)
                return c2

            lax.fori_loop(0, SEQ, grp_body, 0)

        pltpu.sync_copy(rows_v, out_hbm.at[pl.ds(row0, R)])
        return carry

    lax.fori_loop(0, N_CHUNKS, chunk_body, 0)


def kernel(ix, table):
    return _emb_norm(table, ix.astype(jnp.int32))


# speedup vs baseline: 1.3566x; 1.0500x over previous
import functools

import jax
import jax.numpy as jnp
from jax import lax
from jax.experimental import pallas as pl
from jax.experimental.pallas import tpu as pltpu
from jax.experimental.pallas import tpu_sc as plsc

NODE_DIM = 32
SEQ = 50
NC = 2
NS = 16
NW = NC * NS
L = 16

N_ROWS = 16384
ROWS_PER_W = N_ROWS // NW
R = 32
C = R * SEQ
N_CHUNKS = ROWS_PER_W // R


def _rsqrt(s):
    i = plsc.bitcast(s, jnp.int32)
    i = jnp.int32(0x5F3759DF) - (i >> 1)
    y = plsc.bitcast(i, jnp.float32)
    h = 0.5 * s
    y = y * (1.5 - h * y * y)
    y = y * (1.5 - h * y * y)
    return y


_mesh = plsc.VectorSubcoreMesh(core_axis_name="c", subcore_axis_name="s")


@functools.partial(
    pl.kernel,
    mesh=_mesh,
    compiler_params=pltpu.CompilerParams(
        needs_layout_passes=False, use_tc_tiling_on_sc=False
    ),
    out_type=jax.ShapeDtypeStruct((N_ROWS, SEQ, NODE_DIM), jnp.float32),
    scratch_types=[
        pltpu.VMEM((R, SEQ), jnp.int32),
        pltpu.VMEM((C,), jnp.int32),
        pltpu.VMEM((C, NODE_DIM), jnp.float32),
        pltpu.SemaphoreType.DMA,
        pltpu.SemaphoreType.DMA,
    ],
)
def _emb_norm(table_hbm, ix_hbm, out_hbm, idx2d_v, idx_v, rows_v, gsem, osem):
    wid = lax.axis_index("s") * NC + lax.axis_index("c")
    base_row = wid * ROWS_PER_W
    iota = lax.iota(jnp.int32, L)

    def chunk_body(g, carry):
        row0 = base_row + g * R
        pltpu.sync_copy(ix_hbm.at[pl.ds(row0, R)], idx2d_v)

        for i in range(R):
            for c in (0, 16, 32, SEQ - L):
                v = idx2d_v[i, pl.ds(c, L)]
                plsc.store_scatter(idx_v, [i * SEQ + c + iota], v)

        pltpu.async_copy(table_hbm.at[idx_v], rows_v, gsem).wait()

        def grp_body(g2, c2):
            rows16 = g2 * L + iota
            vs = []
            s = None
            for c in range(NODE_DIM):
                cvec = jnp.full((L,), c, jnp.int32)
                v = plsc.load_gather(rows_v, [rows16, cvec])
                vs.append(v)
                sq = v * v
                s = sq if s is None else s + sq
            norm = s * _rsqrt(s)
            inv = 1.0 / (1e-7 + norm)
            for c in range(NODE_DIM):
                cvec = jnp.full((L,), c, jnp.int32)
                plsc.store_scatter(rows_v, [rows16, cvec], vs[c] * inv)
            return c2

        lax.fori_loop(0, C // L, grp_body, 0)

        for j in range(R):
            pltpu.async_copy(
                rows_v.at[pl.ds(j * SEQ, SEQ)], out_hbm.at[row0 + j], osem
            )
        for j in range(R):
            pltpu.make_async_copy(
                rows_v.at[pl.ds(j * SEQ, SEQ)], out_hbm.at[row0 + j], osem
            ).wait()
        return carry

    lax.fori_loop(0, N_CHUNKS, chunk_body, 0)


def kernel(ix, table):
    return _emb_norm(table, ix.astype(jnp.int32))


# speedup vs baseline: 1.3866x; 1.0221x over previous
import functools

import jax
import jax.numpy as jnp
from jax import lax
from jax.experimental import pallas as pl
from jax.experimental.pallas import tpu as pltpu
from jax.experimental.pallas import tpu_sc as plsc

NODE_DIM = 32
SEQ = 50
NC = 2
NS = 16
NW = NC * NS
L = 16

N_ROWS = 16384
ROWS_PER_W = N_ROWS // NW
R = 32
C = R * SEQ
N_CHUNKS = ROWS_PER_W // R
N_PAIRS = N_CHUNKS // 2


def _rsqrt(s):
    i = plsc.bitcast(s, jnp.int32)
    i = jnp.int32(0x5F3759DF) - (i >> 1)
    y = plsc.bitcast(i, jnp.float32)
    h = 0.5 * s
    y = y * (1.5 - h * y * y)
    y = y * (1.5 - h * y * y)
    return y


_mesh = plsc.VectorSubcoreMesh(core_axis_name="c", subcore_axis_name="s")


@functools.partial(
    pl.kernel,
    mesh=_mesh,
    compiler_params=pltpu.CompilerParams(
        needs_layout_passes=False, use_tc_tiling_on_sc=False
    ),
    out_type=jax.ShapeDtypeStruct((N_ROWS, SEQ, NODE_DIM), jnp.float32),
    scratch_types=[
        pltpu.VMEM((R, SEQ), jnp.int32),
        pltpu.VMEM((C,), jnp.int32),
        pltpu.VMEM((C,), jnp.int32),
        pltpu.VMEM((C, NODE_DIM), jnp.float32),
        pltpu.VMEM((C, NODE_DIM), jnp.float32),
        pltpu.SemaphoreType.DMA,
        pltpu.SemaphoreType.DMA,
        pltpu.SemaphoreType.DMA,
        pltpu.SemaphoreType.DMA,
    ],
)
def _emb_norm(
    table_hbm, ix_hbm, out_hbm,
    idx2d_v, idx_a, idx_b, rows_a, rows_b,
    gsem_a, gsem_b, osem_a, osem_b,
):
    wid = lax.axis_index("s") * NC + lax.axis_index("c")
    base_row = wid * ROWS_PER_W
    iota = lax.iota(jnp.int32, L)

    def load_flatten(row0, idx_v):
        pltpu.sync_copy(ix_hbm.at[pl.ds(row0, R)], idx2d_v)
        for i in range(R):
            for c in (0, 16, 32, SEQ - L):
                v = idx2d_v[i, pl.ds(c, L)]
                plsc.store_scatter(idx_v, [i * SEQ + c + iota], v)

    def normalize(rows_v):
        def grp_body(g2, c2):
            rows16 = g2 * L + iota
            vs = []
            s = None
            for c in range(NODE_DIM):
                cvec = jnp.full((L,), c, jnp.int32)
                v = plsc.load_gather(rows_v, [rows16, cvec])
                vs.append(v)
                sq = v * v
                s = sq if s is None else s + sq
            norm = s * _rsqrt(s)
            inv = 1.0 / (1e-7 + norm)
            for c in range(NODE_DIM):
                cvec = jnp.full((L,), c, jnp.int32)
                plsc.store_scatter(rows_v, [rows16, cvec], vs[c] * inv)
            return c2

        lax.fori_loop(0, C // L, grp_body, 0)

    def fire_out(row0, rows_v, osem):
        for j in range(R):
            pltpu.async_copy(
                rows_v.at[pl.ds(j * SEQ, SEQ)], out_hbm.at[row0 + j], osem
            )

    def wait_out(row0, rows_v, osem):
        for j in range(R):
            pltpu.make_async_copy(
                rows_v.at[pl.ds(j * SEQ, SEQ)], out_hbm.at[row0 + j], osem
            ).wait()

    def pair_body(h, carry):
        row_a = base_row + (2 * h) * R
        row_b = row_a + R

        load_flatten(row_a, idx_a)
        cp_a = pltpu.async_copy(table_hbm.at[idx_a], rows_a, gsem_a)
        load_flatten(row_b, idx_b)
        cp_b = pltpu.async_copy(table_hbm.at[idx_b], rows_b, gsem_b)

        cp_a.wait()
        normalize(rows_a)
        fire_out(row_a, rows_a, osem_a)

        cp_b.wait()
        normalize(rows_b)
        fire_out(row_b, rows_b, osem_b)

        wait_out(row_a, rows_a, osem_a)
        wait_out(row_b, rows_b, osem_b)
        return carry

    lax.fori_loop(0, N_PAIRS, pair_body, 0)


def kernel(ix, table):
    return _emb_norm(table, ix.astype(jnp.int32))


# speedup vs baseline: 1.3875x; 1.0007x over previous
import functools

import jax
import jax.numpy as jnp
from jax import lax
from jax.experimental import pallas as pl
from jax.experimental.pallas import tpu as pltpu
from jax.experimental.pallas import tpu_sc as plsc

NODE_DIM = 32
SEQ = 50
NC = 2
NS = 16
NW = NC * NS
L = 16

N_ROWS = 16384
ROWS_PER_W = N_ROWS // NW
R = 32
C = R * SEQ
N_CHUNKS = ROWS_PER_W // R
N_PAIRS = N_CHUNKS // 2


def _rsqrt(s):
    i = plsc.bitcast(s, jnp.int32)
    i = jnp.int32(0x5F3759DF) - (i >> 1)
    y = plsc.bitcast(i, jnp.float32)
    h = 0.5 * s
    y = y * (1.5 - h * y * y)
    y = y * (1.5 - h * y * y)
    return y


_mesh = plsc.VectorSubcoreMesh(core_axis_name="c", subcore_axis_name="s")


@functools.partial(
    pl.kernel,
    mesh=_mesh,
    compiler_params=pltpu.CompilerParams(
        needs_layout_passes=False, use_tc_tiling_on_sc=False
    ),
    out_type=jax.ShapeDtypeStruct((N_ROWS, SEQ, NODE_DIM), jnp.float32),
    scratch_types=[
        pltpu.VMEM((R, SEQ), jnp.int32),
        pltpu.VMEM((C,), jnp.int32),
        pltpu.VMEM((C,), jnp.int32),
        pltpu.VMEM((C, NODE_DIM), jnp.float32),
        pltpu.VMEM((C, NODE_DIM), jnp.float32),
        pltpu.SemaphoreType.DMA,
        pltpu.SemaphoreType.DMA,
        pltpu.SemaphoreType.DMA,
        pltpu.SemaphoreType.DMA,
    ],
)
def _emb_norm(
    table_hbm, ix_hbm, out_hbm,
    idx2d_v, idx_a, idx_b, rows_a, rows_b,
    gsem_a, gsem_b, osem_a, osem_b,
):
    wid = lax.axis_index("s") * NC + lax.axis_index("c")
    base_row = wid * ROWS_PER_W
    iota = lax.iota(jnp.int32, L)

    def load_flatten(row0, idx_v):
        pltpu.sync_copy(ix_hbm.at[pl.ds(row0, R)], idx2d_v)
        for i in range(R):
            for c in (0, 16, 32, SEQ - L):
                v = idx2d_v[i, pl.ds(c, L)]
                plsc.store_scatter(idx_v, [i * SEQ + c + iota], v)

    def normalize(rows_v):
        def grp_body(g2, c2):
            rows16 = g2 * L + iota
            vs = []
            s = None
            for c in range(NODE_DIM):
                cvec = jnp.full((L,), c, jnp.int32)
                v = plsc.load_gather(rows_v, [rows16, cvec])
                vs.append(v)
                sq = v * v
                s = sq if s is None else s + sq
            norm = s * _rsqrt(s)
            inv = 1.0 / (1e-7 + norm)
            for c in range(NODE_DIM):
                cvec = jnp.full((L,), c, jnp.int32)
                plsc.store_scatter(rows_v, [rows16, cvec], vs[c] * inv)
            return c2

        lax.fori_loop(0, C // L, grp_body, 0)

    def fire_out(row0, rows_v, osem):
        for j in range(R):
            pltpu.async_copy(
                rows_v.at[pl.ds(j * SEQ, SEQ)], out_hbm.at[row0 + j], osem
            )

    def wait_out(row0, rows_v, osem):
        for j in range(R):
            pltpu.make_async_copy(
                rows_v.at[pl.ds(j * SEQ, SEQ)], out_hbm.at[row0 + j], osem
            ).wait()

    HC = C // 2

    def start_gather(idx_v, rows_v, gsem):
        c1 = pltpu.async_copy(
            table_hbm.at[idx_v.at[pl.ds(0, HC)]], rows_v.at[pl.ds(0, HC)], gsem
        )
        c2 = pltpu.async_copy(
            table_hbm.at[idx_v.at[pl.ds(HC, HC)]], rows_v.at[pl.ds(HC, HC)], gsem
        )
        return c1, c2

    def pair_body(h, carry):
        row_a = base_row + (2 * h) * R
        row_b = row_a + R

        load_flatten(row_a, idx_a)
        cps_a = start_gather(idx_a, rows_a, gsem_a)
        load_flatten(row_b, idx_b)
        cps_b = start_gather(idx_b, rows_b, gsem_b)

        for cp in cps_a:
            cp.wait()
        normalize(rows_a)
        fire_out(row_a, rows_a, osem_a)

        for cp in cps_b:
            cp.wait()
        normalize(rows_b)
        fire_out(row_b, rows_b, osem_b)

        wait_out(row_a, rows_a, osem_a)
        wait_out(row_b, rows_b, osem_b)
        return carry

    lax.fori_loop(0, N_PAIRS, pair_body, 0)


def kernel(ix, table):
    return _emb_norm(table, ix.astype(jnp.int32))
